# baseline (speedup 1.0000x reference)
import jax
import jax.numpy as jnp
from jax import lax
from jax.experimental import pallas as pl
from jax.experimental.pallas import tpu as pltpu

B = 1024
K = 32000
N0 = 2000
N1 = 1000
CO = 2000
RO = 500
CF = 500
RF = 500

KT = 640
KSTEPS = K // KT


def _ffn_kernel(X_ref, W_ref, bf_ref, Wm_ref, bm_ref, Wc_ref, bc_ref,
                sw_ref, sb_ref, Wr_ref, br_ref, out_ref, acc_ref):
    k = pl.program_id(0)

    @pl.when(k == 0)
    def _init():
        acc_ref[...] = jnp.zeros_like(acc_ref)

    x = X_ref[...].astype(jnp.bfloat16)
    w = W_ref[...].astype(jnp.bfloat16)
    acc_ref[...] += jnp.dot(x, w, preferred_element_type=jnp.float32)

    @pl.when(k == KSTEPS - 1)
    def _epilogue():
        h_in = acc_ref[...] + bf_ref[...]
        h0 = jnp.maximum(h_in, 0.0).astype(jnp.bfloat16)
        h = jnp.dot(h0, Wm_ref[...].astype(jnp.bfloat16),
                    preferred_element_type=jnp.float32) + bm_ref[...]

        col = lax.broadcasted_iota(jnp.int32, (B, N1), 1)
        hc = jnp.where(col < CF, h, 0.0)
        hc = jnp.maximum(hc, 0.0).astype(jnp.bfloat16)
        class_out = jnp.dot(hc, Wc_ref[...].astype(jnp.bfloat16),
                            preferred_element_type=jnp.float32) + bc_ref[...]

        hr = jnp.where(col >= N1 - RF, h, 0.0)
        hr = hr * sw_ref[...] + sb_ref[...]
        hrt = jnp.tanh(hr).astype(jnp.bfloat16)
        reg_out = jnp.dot(hrt, Wr_ref[...].astype(jnp.bfloat16),
                          preferred_element_type=jnp.float32) + br_ref[...]

        out_ref[:, :CO] = class_out
        out_ref[:, CO:] = reg_out


def kernel(X, W_freq, b_freq, Wm, bm, Wc, bc, sw, sb, Wr, br):
    bf2 = b_freq.reshape(1, N0)
    bm2 = bm.reshape(1, N1)
    bc2 = bc.reshape(1, CO)
    sw2 = sw.reshape(1, N1)
    sb2 = sb.reshape(1, N1)
    br2 = br.reshape(1, RO)

    full = lambda shape: pl.BlockSpec(shape, lambda k: (0,) * len(shape))
    out = pl.pallas_call(
        _ffn_kernel,
        grid=(KSTEPS,),
        in_specs=[
            pl.BlockSpec((B, KT), lambda k: (0, k)),
            pl.BlockSpec((KT, N0), lambda k: (k, 0)),
            full((1, N0)),
            full((N0, N1)),
            full((1, N1)),
            full((N1, CO)),
            full((1, CO)),
            full((1, N1)),
            full((1, N1)),
            full((N1, RO)),
            full((1, RO)),
        ],
        out_specs=full((B, CO + RO)),
        out_shape=jax.ShapeDtypeStruct((B, CO + RO), jnp.float32),
        scratch_shapes=[pltpu.VMEM((B, N0), jnp.float32)],
        compiler_params=pltpu.CompilerParams(
            dimension_semantics=("arbitrary",),
        ),
    )(X, W_freq, bf2, Wm, bm2, Wc, bc2, sw2, sb2, Wr, br2)
    return out


# speedup vs baseline: 1.0640x; 1.0640x over previous
import jax
import jax.numpy as jnp
from jax import lax
from jax.experimental import pallas as pl
from jax.experimental.pallas import tpu as pltpu

B = 1024
K = 32000
N0 = 2000
N1 = 1000
CO = 2000
RO = 500
CF = 500
RF = 500

KT = 1280
KSTEPS = K // KT

_DEF = lax.Precision.DEFAULT


def _matmul_kernel(X_ref, W_ref, bf_ref, h0_ref, acc_ref):
    k = pl.program_id(0)

    @pl.when(k == 0)
    def _init():
        acc_ref[...] = jnp.zeros_like(acc_ref)

    acc_ref[...] += jnp.dot(X_ref[...], W_ref[...],
                            preferred_element_type=jnp.float32,
                            precision=_DEF)

    @pl.when(k == KSTEPS - 1)
    def _bias_relu():
        h0_ref[...] = jnp.maximum(acc_ref[...] + bf_ref[...], 0.0)


def _heads_kernel(h0_ref, Wm_ref, bm_ref, Wc_ref, bc_ref,
                  sw_ref, sb_ref, Wr_ref, br_ref, out_ref):
    h = jnp.dot(h0_ref[...], Wm_ref[...], preferred_element_type=jnp.float32,
                precision=_DEF) + bm_ref[...]

    col = lax.broadcasted_iota(jnp.int32, (B, N1), 1)
    hc = jnp.maximum(jnp.where(col < CF, h, 0.0), 0.0)
    class_out = jnp.dot(hc, Wc_ref[...], preferred_element_type=jnp.float32,
                        precision=_DEF) + bc_ref[...]

    hr = jnp.where(col >= N1 - RF, h, 0.0) * sw_ref[...] + sb_ref[...]
    hrt = jnp.tanh(hr)
    reg_out = jnp.dot(hrt, Wr_ref[...], preferred_element_type=jnp.float32,
                      precision=_DEF) + br_ref[...]

    out_ref[:, :CO] = class_out
    out_ref[:, CO:] = reg_out


def _full(shape):
    return pl.BlockSpec(shape, lambda *args: (0,) * len(shape))


def kernel(X, W_freq, b_freq, Wm, bm, Wc, bc, sw, sb, Wr, br):
    bf2 = b_freq.reshape(1, N0)
    bm2 = bm.reshape(1, N1)
    bc2 = bc.reshape(1, CO)
    sw2 = sw.reshape(1, N1)
    sb2 = sb.reshape(1, N1)
    br2 = br.reshape(1, RO)

    h0 = pl.pallas_call(
        _matmul_kernel,
        grid=(KSTEPS,),
        in_specs=[
            pl.BlockSpec((B, KT), lambda k: (0, k)),
            pl.BlockSpec((KT, N0), lambda k: (k, 0)),
            _full((1, N0)),
        ],
        out_specs=_full((B, N0)),
        out_shape=jax.ShapeDtypeStruct((B, N0), jnp.float32),
        scratch_shapes=[pltpu.VMEM((B, N0), jnp.float32)],
        compiler_params=pltpu.CompilerParams(
            dimension_semantics=("arbitrary",),
        ),
    )(X, W_freq, bf2)

    out = pl.pallas_call(
        _heads_kernel,
        in_specs=[
            _full((B, N0)),
            _full((N0, N1)),
            _full((1, N1)),
            _full((N1, CO)),
            _full((1, CO)),
            _full((1, N1)),
            _full((1, N1)),
            _full((N1, RO)),
            _full((1, RO)),
        ],
        out_specs=_full((B, CO + RO)),
        out_shape=jax.ShapeDtypeStruct((B, CO + RO), jnp.float32),
    )(h0, Wm, bm2, Wc, bc2, sw2, sb2, Wr, br2)
    return out
